# baseline (speedup 1.0000x reference)
import functools

import jax
import jax.numpy as jnp
from jax import lax
from jax.experimental import pallas as pl
from jax.experimental.pallas import tpu as pltpu
from jax.experimental.pallas import tpu_sc as plsc


def _make_sc_gather(entity_dim: int, batch: int):
    info = plsc.get_sparse_core_info()
    nw = info.num_cores * info.num_subcores
    assert batch % nw == 0
    b_per_w = batch // nw
    lanes = info.num_lanes

    mesh = plsc.VectorSubcoreMesh(core_axis_name="c", subcore_axis_name="s")

    @functools.partial(
        pl.kernel,
        mesh=mesh,
        out_type=jax.ShapeDtypeStruct((batch, 128), jnp.float32),
        scratch_types=[
            pltpu.VMEM((batch,), jnp.int32),
            pltpu.VMEM((lanes, entity_dim, 128), jnp.float32),
            pltpu.VMEM((b_per_w, 128), jnp.float32),
            pltpu.SemaphoreType.DMA,
        ],
        compiler_params=pltpu.CompilerParams(needs_layout_passes=False),
    )
    def gather_kernel(table_hbm, idx_hbm, out_hbm, idx_v, blk_v, out_v, sem):
        wid = lax.axis_index("s") * info.num_cores + lax.axis_index("c")
        base = wid * b_per_w
        pltpu.sync_copy(idx_hbm, idx_v)
        for w in range(b_per_w // lanes):
            iv = idx_v[pl.ds(base + w * lanes, lanes)]
            ctv = (iv >> 7) << 7
            copies = []
            for jj in range(lanes):
                off = pl.multiple_of(ctv[jj], 128)
                copies.append(
                    pltpu.async_copy(
                        table_hbm.at[:, pl.ds(off, 128)], blk_v.at[jj], sem
                    )
                )
            for c in copies:
                c.wait()
            cov = iv & 127
            jv_local = lax.iota(jnp.int32, lanes)
            jv_out = jv_local + w * lanes
            for k in range(entity_dim):
                kv = jnp.full((lanes,), k, jnp.int32)
                vals = plsc.load_gather(blk_v, [jv_local, kv, cov])
                plsc.store_scatter(out_v, [jv_out, kv], vals)
        pltpu.sync_copy(out_v, out_hbm.at[pl.ds(base, b_per_w)])

    return gather_kernel


_TC_K = 8


def _tc_gather_body(idx_ref, *refs):
    i = pl.program_id(0)
    blocks = refs[:_TC_K]
    o_ref = refs[_TC_K]
    oh_rows = []
    for k in range(_TC_K):
        c = idx_ref[i * _TC_K + k] & 127
        oh_rows.append(
            (lax.broadcasted_iota(jnp.int32, (1, 128), 1) == c).astype(jnp.float32)
        )
    oh = jnp.concatenate(oh_rows, axis=0)
    stacked = jnp.stack([b[...] for b in blocks], axis=0)
    sel = lax.dot_general(
        oh, stacked, (((1,), (2,)), ((0,), (0,))), preferred_element_type=jnp.float32
    )
    o_ref[...] = jnp.pad(sel, ((0, 0), (0, 96)))


def _make_tc_gather(batch: int):
    grid = batch // _TC_K

    def _blk_spec(k):
        return pl.BlockSpec(
            (32, 128), lambda i, idx_ref, _k=k: (0, idx_ref[i * _TC_K + _k] >> 7)
        )

    return pl.pallas_call(
        _tc_gather_body,
        grid_spec=pltpu.PrefetchScalarGridSpec(
            num_scalar_prefetch=1,
            grid=(grid,),
            in_specs=[_blk_spec(k) for k in range(_TC_K)],
            out_specs=pl.BlockSpec((_TC_K, 128), lambda i, idx_ref: (i, 0)),
        ),
        out_shape=jax.ShapeDtypeStruct((batch, 128), jnp.float32),
    )


def _project_body(ga_ref, gb_ref, w_ref, b_ref, o_ref):
    g = jnp.concatenate([ga_ref[:, :32], gb_ref[:, :32]], axis=0)
    o_ref[...] = (
        lax.dot_general(
            w_ref[...], g, (((0,), (1,)), ((), ())),
            preferred_element_type=jnp.float32,
        )
        + b_ref[...]
    )


def kernel(x, entity_bank, W, b):
    batch = x.shape[0]
    num_entities, entity_dim = entity_bank.shape
    out_dim = W.shape[1]
    half = batch // 2

    idx = x.reshape(batch).astype(jnp.int32)
    table_t = entity_bank.T
    gathered_sc = _make_sc_gather(entity_dim, half)(table_t, idx[:half])
    gathered_tc = _make_tc_gather(half)(
        idx[half:], *([table_t] * _TC_K)
    )

    out_t = pl.pallas_call(
        _project_body,
        out_shape=jax.ShapeDtypeStruct((out_dim, batch), jnp.float32),
    )(gathered_sc, gathered_tc, W, b.reshape(out_dim, 1))
    return out_t.T


# speedup vs baseline: 1.8811x; 1.8811x over previous
import functools

import jax
import jax.numpy as jnp
from jax import lax
from jax.experimental import pallas as pl
from jax.experimental.pallas import tpu as pltpu
from jax.experimental.pallas import tpu_sc as plsc


def _make_sc_gather(entity_dim: int, batch: int):
    info = plsc.get_sparse_core_info()
    nw = info.num_cores * info.num_subcores
    assert batch % nw == 0
    b_per_w = batch // nw
    lanes = info.num_lanes

    mesh = plsc.VectorSubcoreMesh(core_axis_name="c", subcore_axis_name="s")

    @functools.partial(
        pl.kernel,
        mesh=mesh,
        out_type=jax.ShapeDtypeStruct((batch, 128), jnp.float32),
        scratch_types=[
            pltpu.VMEM((batch,), jnp.int32),
            pltpu.VMEM((lanes, entity_dim, 128), jnp.float32),
            pltpu.VMEM((b_per_w, 128), jnp.float32),
            pltpu.SemaphoreType.DMA,
        ],
        compiler_params=pltpu.CompilerParams(
            needs_layout_passes=False, skip_device_barrier=True
        ),
    )
    def gather_kernel(table_hbm, idx_hbm, out_hbm, idx_v, blk_v, out_v, sem):
        wid = lax.axis_index("s") * info.num_cores + lax.axis_index("c")
        base = wid * b_per_w
        pltpu.sync_copy(idx_hbm, idx_v)
        for w in range(b_per_w // lanes):
            iv = idx_v[pl.ds(base + w * lanes, lanes)]
            ctv = (iv >> 7) << 7
            copies = []
            for jj in range(lanes):
                off = pl.multiple_of(ctv[jj], 128)
                copies.append(
                    pltpu.async_copy(
                        table_hbm.at[:, pl.ds(off, 128)], blk_v.at[jj], sem
                    )
                )
            for c in copies:
                c.wait()
            cov = iv & 127
            jv_local = lax.iota(jnp.int32, lanes)
            jv_out = jv_local + w * lanes
            for k in range(entity_dim):
                kv = jnp.full((lanes,), k, jnp.int32)
                vals = plsc.load_gather(blk_v, [jv_local, kv, cov])
                plsc.store_scatter(out_v, [jv_out, kv], vals)
        pltpu.sync_copy(out_v, out_hbm.at[pl.ds(base, b_per_w)])

    return gather_kernel


def _project_body(g_ref, w_ref, b_ref, o_ref):
    o_ref[...] = (
        lax.dot_general(
            w_ref[...],
            g_ref[:, :32],
            (((0,), (1,)), ((), ())),
            preferred_element_type=jnp.float32,
        )
        + b_ref[...]
    )


def kernel(x, entity_bank, W, b):
    batch = x.shape[0]
    num_entities, entity_dim = entity_bank.shape
    out_dim = W.shape[1]

    idx = x.reshape(batch).astype(jnp.int32)
    gathered = _make_sc_gather(entity_dim, batch)(entity_bank.T, idx)

    out_t = pl.pallas_call(
        _project_body,
        out_shape=jax.ShapeDtypeStruct((out_dim, batch), jnp.float32),
    )(gathered, W, b.reshape(out_dim, 1))
    return out_t.T


# speedup vs baseline: 1.9158x; 1.0185x over previous
import functools

import jax
import jax.numpy as jnp
from jax import lax
from jax.experimental import pallas as pl
from jax.experimental.pallas import tpu as pltpu
from jax.experimental.pallas import tpu_sc as plsc


def _make_sc_gather(entity_dim: int, batch: int):
    info = plsc.get_sparse_core_info()
    nw = info.num_cores * info.num_subcores
    assert batch % nw == 0
    b_per_w = batch // nw
    lanes = info.num_lanes

    mesh = plsc.VectorSubcoreMesh(core_axis_name="c", subcore_axis_name="s")

    @functools.partial(
        pl.kernel,
        mesh=mesh,
        out_type=jax.ShapeDtypeStruct((batch, 128), jnp.float32),
        scratch_types=[
            pltpu.VMEM((batch,), jnp.int32),
            pltpu.VMEM((lanes, entity_dim, 128), jnp.float32),
            pltpu.VMEM((b_per_w, 128), jnp.float32),
            pltpu.SemaphoreType.DMA,
        ],
        compiler_params=pltpu.CompilerParams(needs_layout_passes=False),
    )
    def gather_kernel(table_hbm, idx_hbm, out_hbm, idx_v, blk_v, out_v, sem):
        wid = lax.axis_index("s") * info.num_cores + lax.axis_index("c")
        base = wid * b_per_w
        pltpu.sync_copy(idx_hbm, idx_v)
        for w in range(b_per_w // lanes):
            iv = idx_v[pl.ds(base + w * lanes, lanes)]
            ctv = (iv >> 7) << 7
            copies = []
            for jj in range(lanes):
                off = pl.multiple_of(ctv[jj], 128)
                copies.append(
                    pltpu.async_copy(
                        table_hbm.at[:, pl.ds(off, 128)], blk_v.at[jj], sem
                    )
                )
            for c in copies:
                c.wait()
            cov = iv & 127
            jv_local = lax.iota(jnp.int32, lanes)
            jv_out = jv_local + w * lanes
            for k in range(entity_dim):
                kv = jnp.full((lanes,), k, jnp.int32)
                vals = plsc.load_gather(blk_v, [jv_local, kv, cov])
                plsc.store_scatter(out_v, [jv_out, kv], vals)
        pltpu.sync_copy(out_v, out_hbm.at[pl.ds(base, b_per_w)])

    return gather_kernel


_CHUNK = 64


def _tc_gather_body(idx_sm, idx_v, table_hbm, o_ref, blk_v, sem):
    n = o_ref.shape[0]
    nchunks = n // _CHUNK

    def fire(c):
        for j in range(_CHUNK):
            i = c * _CHUNK + j
            off = pl.multiple_of((idx_sm[i] >> 7) << 7, 128)
            pltpu.make_async_copy(
                table_hbm.at[:, pl.ds(off, 128)], blk_v.at[i], sem
            ).start()

    def drain_select(c):
        for j in range(_CHUNK):
            i = c * _CHUNK + j
            pltpu.make_async_copy(
                table_hbm.at[:, pl.ds(0, 128)], blk_v.at[i], sem
            ).wait()
        sl = pl.ds(c * _CHUNK, _CHUNK)
        cov = idx_v[sl] & 127
        oh = (
            lax.broadcasted_iota(jnp.int32, (_CHUNK, 128), 1) == cov
        ).astype(jnp.float32)
        sel = lax.dot_general(
            blk_v[sl],
            oh,
            (((2,), (1,)), ((0,), (0,))),
            preferred_element_type=jnp.float32,
        )
        o_ref[sl, :] = sel

    fire(0)
    for c in range(nchunks):
        if c + 1 < nchunks:
            fire(c + 1)
        drain_select(c)


def _make_tc_gather(entity_dim: int, batch: int):
    return pl.pallas_call(
        _tc_gather_body,
        in_specs=[
            pl.BlockSpec(memory_space=pltpu.SMEM),
            pl.BlockSpec(memory_space=pltpu.VMEM),
            pl.BlockSpec(memory_space=pl.ANY),
        ],
        out_specs=pl.BlockSpec(memory_space=pltpu.VMEM),
        out_shape=jax.ShapeDtypeStruct((batch, entity_dim), jnp.float32),
        scratch_shapes=[
            pltpu.VMEM((batch, entity_dim, 128), jnp.float32),
            pltpu.SemaphoreType.DMA,
        ],
    )


def _project_body(ga_ref, gb_ref, w_ref, b_ref, o_ref):
    g = jnp.concatenate([ga_ref[:, :32], gb_ref[...]], axis=0)
    o_ref[...] = (
        lax.dot_general(
            w_ref[...], g, (((0,), (1,)), ((), ())),
            preferred_element_type=jnp.float32,
        )
        + b_ref[...]
    )


def kernel(x, entity_bank, W, b):
    batch = x.shape[0]
    num_entities, entity_dim = entity_bank.shape
    out_dim = W.shape[1]
    half = batch // 2

    idx = x.reshape(batch).astype(jnp.int32)
    table_t = entity_bank.T
    gathered_sc = _make_sc_gather(entity_dim, half)(table_t, idx[:half])
    gathered_tc = _make_tc_gather(entity_dim, half)(
        idx[half:], idx[half:].reshape(half, 1), table_t
    )

    out_t = pl.pallas_call(
        _project_body,
        out_shape=jax.ShapeDtypeStruct((out_dim, batch), jnp.float32),
    )(gathered_sc, gathered_tc, W, b.reshape(out_dim, 1))
    return out_t.T
